# baseline (speedup 1.0000x reference)
import functools

import jax
import jax.numpy as jnp
from jax import lax
from jax.experimental import pallas as pl
from jax.experimental.pallas import tpu as pltpu
from jax.experimental.pallas import tpu_sc as plsc

_B, _S, _EMB = 4, 2048, 768
_NW = 32
_SPW = _S // _NW
_SUB = 32
_NPH = _SPW // _SUB
_STEPS = _B * _NPH
_LANES = _EMB // 16
_NBUF = 4
_PREFETCH = _NBUF - 2

_mesh = plsc.VectorSubcoreMesh(core_axis_name="c", subcore_axis_name="s")


@functools.partial(
    pl.kernel,
    out_type=jax.ShapeDtypeStruct((_B, _S, _EMB), jnp.float32),
    mesh=_mesh,
    scratch_types=[
        pltpu.VMEM((_B * _SPW,), jnp.int32),
        pltpu.VMEM((_SUB, _EMB), jnp.float32),
        [pltpu.VMEM((_SUB, _EMB), jnp.float32)] * _NBUF,
        [pltpu.SemaphoreType.DMA] * _NBUF,
        [pltpu.SemaphoreType.DMA] * _NBUF,
        pltpu.SemaphoreType.DMA,
        pltpu.SemaphoreType.DMA,
    ],
)
def _embed(x_hbm, tok_hbm, pos_hbm, out_hbm,
           idx_v, pos_v, bufs, gsems, ssems, psem, isem):
    wid = lax.axis_index("s") * 2 + lax.axis_index("c")
    s_base = wid * _SPW

    def pos_src(phase):
        return pos_hbm.at[pl.ds(s_base + phase * _SUB, _SUB), :]

    pltpu.async_copy(pos_src(0), pos_v, psem)

    def x_slice(k):
        phase, b = divmod(k, _B)
        return x_hbm.at[pl.ds(b * _S + s_base + phase * _SUB, _SUB)]

    def idx_slice(k):
        return idx_v.at[pl.ds(k * _SUB, _SUB)]

    for k in range(_STEPS):
        pltpu.async_copy(x_slice(k), idx_slice(k), isem)
    for k in range(_STEPS):
        pltpu.make_async_copy(x_slice(k), idx_slice(k), isem).wait()

    def out_slice(k):
        phase, b = divmod(k, _B)
        return out_hbm.at[b, pl.ds(s_base + phase * _SUB, _SUB), :]

    def gather_start(k):
        pltpu.async_copy(tok_hbm.at[idx_slice(k)], bufs[k % _NBUF], gsems[k % _NBUF])

    def gather_wait(k):
        pltpu.make_async_copy(
            tok_hbm.at[idx_slice(k)], bufs[k % _NBUF], gsems[k % _NBUF]
        ).wait()

    def store_start(k):
        pltpu.async_copy(bufs[k % _NBUF], out_slice(k), ssems[k % _NBUF])

    def store_wait(k):
        pltpu.make_async_copy(bufs[k % _NBUF], out_slice(k), ssems[k % _NBUF]).wait()

    for k in range(_PREFETCH):
        gather_start(k)
    for k in range(_STEPS):
        if k + _PREFETCH < _STEPS:
            if k + _PREFETCH - _NBUF >= 0:
                store_wait(k + _PREFETCH - _NBUF)
            gather_start(k + _PREFETCH)
        gather_wait(k)
        if k == 0 or k == _B:
            pltpu.make_async_copy(pos_src(k // _B), pos_v, psem).wait()

        buf = bufs[k % _NBUF]

        @plsc.parallel_loop(0, _SUB, unroll=2)
        def _(r):
            for j in range(_LANES):
                col = pl.ds(j * 16, 16)
                plsc.addupdate(buf.at[r, col], pos_v[r, col])
        if k == _B - 1:
            pltpu.async_copy(pos_src(1), pos_v, psem)
        store_start(k)

    for k in range(_STEPS - _NBUF, _STEPS):
        store_wait(k)


def kernel(x, tok_emb, pos_emb):
    return _embed(x.reshape(_B * _S).astype(jnp.int32), tok_emb, pos_emb)


# speedup vs baseline: 1.0629x; 1.0629x over previous
import functools

import jax
import jax.numpy as jnp
from jax import lax
from jax.experimental import pallas as pl
from jax.experimental.pallas import tpu as pltpu
from jax.experimental.pallas import tpu_sc as plsc

_B, _S, _EMB = 4, 2048, 768
_NW = 32
_SPW = _S // _NW
_SUB = 32
_NPH = _SPW // _SUB
_STEPS = _B * _NPH
_LANES = _EMB // 16
_NBUF = 4
_PREFETCH = _NBUF - 2

_mesh = plsc.VectorSubcoreMesh(core_axis_name="c", subcore_axis_name="s")


@functools.partial(
    pl.kernel,
    out_type=jax.ShapeDtypeStruct((_B, _S, _EMB), jnp.float32),
    mesh=_mesh,
    scratch_types=[
        pltpu.VMEM((_B * _SPW,), jnp.int32),
        pltpu.VMEM((_SUB, _EMB), jnp.float32),
        [pltpu.VMEM((_SUB, _EMB), jnp.float32)] * _NBUF,
        [pltpu.SemaphoreType.DMA] * _NBUF,
        [pltpu.SemaphoreType.DMA] * _NBUF,
        pltpu.SemaphoreType.DMA,
        pltpu.SemaphoreType.DMA,
    ],
)
def _embed(x_hbm, tok_hbm, pos_hbm, out_hbm,
           idx_v, pos_v, bufs, gsems, ssems, psem, isem):
    wid = lax.axis_index("s") * 2 + lax.axis_index("c")
    s_base = wid * _SPW

    def pos_src(phase):
        return pos_hbm.at[pl.ds(s_base + phase * _SUB, _SUB), :]

    pltpu.async_copy(pos_src(0), pos_v, psem)

    def x_slice(k):
        phase, b = divmod(k, _B)
        return x_hbm.at[pl.ds(b * _S + s_base + phase * _SUB, _SUB)]

    def idx_slice(k):
        return idx_v.at[pl.ds(k * _SUB, _SUB)]

    pltpu.sync_copy(x_slice(0), idx_slice(0))
    for k in range(1, _STEPS):
        pltpu.async_copy(x_slice(k), idx_slice(k), isem)

    def out_slice(k):
        phase, b = divmod(k, _B)
        return out_hbm.at[b, pl.ds(s_base + phase * _SUB, _SUB), :]

    def gather_start(k):
        pltpu.async_copy(tok_hbm.at[idx_slice(k)], bufs[k % _NBUF], gsems[k % _NBUF])

    def gather_wait(k):
        pltpu.make_async_copy(
            tok_hbm.at[idx_slice(k)], bufs[k % _NBUF], gsems[k % _NBUF]
        ).wait()

    def store_start(k):
        pltpu.async_copy(bufs[k % _NBUF], out_slice(k), ssems[k % _NBUF])

    def store_wait(k):
        pltpu.make_async_copy(bufs[k % _NBUF], out_slice(k), ssems[k % _NBUF]).wait()

    gather_start(0)
    for k in range(1, _STEPS):
        pltpu.make_async_copy(x_slice(k), idx_slice(k), isem).wait()
    for k in range(1, _PREFETCH):
        gather_start(k)
    for k in range(_STEPS):
        if k + _PREFETCH < _STEPS:
            if k + _PREFETCH - _NBUF >= 0:
                store_wait(k + _PREFETCH - _NBUF)
            gather_start(k + _PREFETCH)
        gather_wait(k)
        if k == 0 or k == _B:
            pltpu.make_async_copy(pos_src(k // _B), pos_v, psem).wait()

        buf = bufs[k % _NBUF]

        def add_row(r, _):
            for j in range(_LANES):
                col = pl.ds(j * 16, 16)
                plsc.addupdate(buf.at[r, col], pos_v[r, col])
            return 0

        lax.fori_loop(0, _SUB, add_row, 0)
        if k == _B - 1:
            pltpu.async_copy(pos_src(1), pos_v, psem)
        store_start(k)

    for k in range(_STEPS - _NBUF, _STEPS):
        store_wait(k)


def kernel(x, tok_emb, pos_emb):
    return _embed(x.reshape(_B * _S).astype(jnp.int32), tok_emb, pos_emb)
